# Initial kernel scaffold; baseline (speedup 1.0000x reference)
import jax
import jax.numpy as jnp
from jax.experimental import pallas as pl


def kernel(x, table, w_lin, bias):
    raise NotImplementedError("write your pallas kernel here")


# speedup vs baseline: 1.2096x; 1.2096x over previous
import functools

import jax
import jax.numpy as jnp
from jax import lax
from jax.experimental import pallas as pl
from jax.experimental.pallas import tpu as pltpu
from jax.experimental.pallas import tpu_sc as plsc

F = 26
D = 16
FIELD_DIM = 40000


def _fm_body(idx_hbm, table_hbm, w_hbm, bias_hbm, out_hbm,
             idx_v, rows_v, wv_v, bias_v, out_v, sem,
             *, nchunk, csize, samp_per_tile, num_cores):
    wid = lax.axis_index("s") * num_cores + lax.axis_index("c")

    pltpu.sync_copy(idx_hbm.at[wid], idx_v)
    pltpu.sync_copy(bias_hbm, bias_v)

    lanes = jnp.arange(D, dtype=jnp.int32)

    @pl.loop(0, nchunk)
    def _chunk(c):
        copies = []
        for f in range(F):
            r = f * nchunk + c
            copies.append(
                pltpu.async_copy(table_hbm.at[idx_v.at[r]],
                                 rows_v.at[pl.ds(f * csize, csize)], sem))
            copies.append(
                pltpu.async_copy(w_hbm.at[idx_v.at[r]], wv_v.at[f], sem))
        for cp in copies:
            cp.wait()

        @pl.loop(0, csize // D)
        def _group(g):
            j0 = g * D
            jvec = lanes + j0

            lin = bias_v[...]
            for f in range(F):
                lin = lin + wv_v[f, pl.ds(j0, D)]

            def dim_step(d, t):
                dvec = jnp.full((D,), d, dtype=jnp.int32)
                s = jnp.zeros((D,), jnp.float32)
                ssq = jnp.zeros((D,), jnp.float32)
                for f in range(F):
                    rv = plsc.load_gather(rows_v, [jvec + (f * csize), dvec])
                    s = s + rv
                    ssq = ssq + rv * rv
                return t + (s * s - ssq)

            t = lax.fori_loop(0, D, dim_step, jnp.zeros((D,), jnp.float32))
            z = lin + 0.5 * t
            res = 1.0 / (1.0 + jnp.exp(-z))
            out_v[pl.ds(c * csize + j0, D)] = res

    pltpu.sync_copy(out_v, out_hbm.at[pl.ds(wid * samp_per_tile, samp_per_tile)])


def kernel(x, table, w_lin, bias):
    B, nf = x.shape
    assert nf == F
    info = plsc.get_sparse_core_info()
    nc, ns = info.num_cores, info.num_subcores
    nw = nc * ns
    samp = B // nw
    csize = 128
    nchunk = samp // csize

    offs = (jnp.arange(F, dtype=jnp.int32) * FIELD_DIM)
    idx = x.astype(jnp.int32) + offs[None, :]
    idx_r = (idx.reshape(nw, nchunk, csize, F)
                .transpose(0, 3, 1, 2)
                .reshape(nw, F * nchunk, csize))
    w_flat = w_lin.reshape(-1)
    bias_splat = jnp.broadcast_to(bias.astype(jnp.float32), (D,))

    mesh = plsc.VectorSubcoreMesh(core_axis_name="c", subcore_axis_name="s",
                                  num_cores=nc, num_subcores=ns)
    body = functools.partial(_fm_body, nchunk=nchunk, csize=csize,
                             samp_per_tile=samp, num_cores=nc)
    run = pl.kernel(
        body,
        out_type=jax.ShapeDtypeStruct((B,), jnp.float32),
        mesh=mesh,
        scratch_types=[
            pltpu.VMEM((F * nchunk, csize), jnp.int32),
            pltpu.VMEM((F * csize, D), jnp.float32),
            pltpu.VMEM((F, csize), jnp.float32),
            pltpu.VMEM((D,), jnp.float32),
            pltpu.VMEM((samp,), jnp.float32),
            pltpu.SemaphoreType.DMA,
        ],
        compiler_params=pltpu.CompilerParams(needs_layout_passes=False,
                                             use_tc_tiling_on_sc=False),
    )
    return run(idx_r, table, w_flat, bias_splat)
